# baseline (speedup 1.0000x reference)
import functools

import jax
import jax.numpy as jnp
from jax import lax
from jax.experimental import pallas as pl
from jax.experimental.pallas import tpu as pltpu
from jax.experimental.pallas import tpu_sc as plsc

_NBUF = 4


def _build_gather(B, H, V, D, NC, NS):
    NW = NC * NS
    RB = B // NW
    NI = RB // _NBUF
    mesh = plsc.VectorSubcoreMesh(core_axis_name="c", subcore_axis_name="s")

    @functools.partial(
        pl.kernel,
        mesh=mesh,
        compiler_params=pltpu.CompilerParams(use_tc_tiling_on_sc=False),
        out_type=jax.ShapeDtypeStruct((B * H, 2 * D), jnp.float32),
        scratch_types=[
            pltpu.VMEM((RB, H), jnp.int32),
            [pltpu.VMEM((H, 2 * D), jnp.float32) for _ in range(_NBUF)],
            [pltpu.SemaphoreType.DMA for _ in range(_NBUF)],
            [pltpu.SemaphoreType.DMA for _ in range(_NBUF)],
        ],
    )
    def gather_kernel(table_hbm, idx_hbm, out_hbm, idx_v, bufs, gsems, ssems):
        wid = lax.axis_index("s") * NC + lax.axis_index("c")
        base = wid * RB
        pltpu.sync_copy(idx_hbm.at[pl.ds(base, RB)], idx_v)

        def start_g(b, r):
            pltpu.async_copy(table_hbm.at[idx_v.at[r]], bufs[b], gsems[b])

        def wait_g(b, r):
            pltpu.make_async_copy(
                table_hbm.at[idx_v.at[r]], bufs[b], gsems[b]
            ).wait()

        def start_s(b, r):
            pltpu.async_copy(
                bufs[b], out_hbm.at[pl.ds((base + r) * H, H)], ssems[b]
            )

        def wait_s(b, r):
            pltpu.make_async_copy(
                bufs[b], out_hbm.at[pl.ds((base + r) * H, H)], ssems[b]
            ).wait()

        start_g(0, 0)
        start_g(1, 1)

        def body(i, carry):
            r0 = i * _NBUF
            for b in range(_NBUF):
                r = r0 + b
                pb = (b + 2) % _NBUF
                pr = r + 2
                if b < 2:
                    @pl.when(i > 0)
                    def _():
                        wait_s(pb, pr - _NBUF)
                        start_g(pb, pr)

                    @pl.when(i == 0)
                    def _():
                        start_g(pb, pr)
                else:
                    wait_s(pb, pr - _NBUF)

                    @pl.when(i < NI - 1)
                    def _():
                        start_g(pb, pr)
                wait_g(b, r)
                start_s(b, r)
            return carry

        lax.fori_loop(0, NI, body, 0)
        wait_s(2, RB - 2)
        wait_s(3, RB - 1)

    return gather_kernel


def kernel(indices, table):
    B, H = indices.shape
    V, D = table.shape
    info = plsc.get_sparse_core_info()
    gather = _build_gather(B, H, V, D, info.num_cores, info.num_subcores)
    table_padded = jnp.pad(table, ((0, 0), (0, D)))
    padded = gather(table_padded, indices.astype(jnp.int32))
    return padded[:, :D].reshape(B, H, D)


# speedup vs baseline: 1.0877x; 1.0877x over previous
import functools

import jax
import jax.numpy as jnp
from jax import lax
from jax.experimental import pallas as pl
from jax.experimental.pallas import tpu as pltpu
from jax.experimental.pallas import tpu_sc as plsc

_NBUF = 4


def _build_gather(B, H, V, D, NC, NS):
    NW = NC * NS
    RB = B // NW
    NI = RB // _NBUF
    mesh = plsc.VectorSubcoreMesh(core_axis_name="c", subcore_axis_name="s")

    @functools.partial(
        pl.kernel,
        mesh=mesh,
        compiler_params=pltpu.CompilerParams(use_tc_tiling_on_sc=False),
        out_type=jax.ShapeDtypeStruct((B * H, 2 * D), jnp.float32),
        scratch_types=[
            pltpu.VMEM((RB, H), jnp.int32),
            [pltpu.VMEM((H, D), jnp.float32) for _ in range(_NBUF)],
            [pltpu.SemaphoreType.DMA for _ in range(_NBUF)],
            [pltpu.SemaphoreType.DMA for _ in range(_NBUF)],
        ],
    )
    def gather_kernel(table_hbm, idx_hbm, out_hbm, idx_v, bufs, gsems, ssems):
        wid = lax.axis_index("s") * NC + lax.axis_index("c")
        base = wid * RB
        pltpu.sync_copy(idx_hbm.at[pl.ds(base, RB)], idx_v)

        def start_g(b, r):
            pltpu.async_copy(table_hbm.at[idx_v.at[r]], bufs[b], gsems[b])

        def wait_g(b, r):
            pltpu.make_async_copy(
                table_hbm.at[idx_v.at[r]], bufs[b], gsems[b]
            ).wait()

        def start_s(b, r):
            pltpu.async_copy(
                bufs[b],
                out_hbm.at[pl.ds((base + r) * H, H), pl.ds(0, D)],
                ssems[b],
            )

        def wait_s(b, r):
            pltpu.make_async_copy(
                bufs[b],
                out_hbm.at[pl.ds((base + r) * H, H), pl.ds(0, D)],
                ssems[b],
            ).wait()

        start_g(0, 0)
        start_g(1, 1)

        def body(i, carry):
            r0 = i * _NBUF
            for b in range(_NBUF):
                r = r0 + b
                pb = (b + 2) % _NBUF
                pr = r + 2
                if b < 2:
                    @pl.when(i > 0)
                    def _():
                        wait_s(pb, pr - _NBUF)
                        start_g(pb, pr)

                    @pl.when(i == 0)
                    def _():
                        start_g(pb, pr)
                else:
                    wait_s(pb, pr - _NBUF)

                    @pl.when(i < NI - 1)
                    def _():
                        start_g(pb, pr)
                wait_g(b, r)
                start_s(b, r)
            return carry

        lax.fori_loop(0, NI, body, 0)
        wait_s(2, RB - 2)
        wait_s(3, RB - 1)

    return gather_kernel


def kernel(indices, table):
    B, H = indices.shape
    V, D = table.shape
    info = plsc.get_sparse_core_info()
    gather = _build_gather(B, H, V, D, info.num_cores, info.num_subcores)
    padded = gather(table, indices.astype(jnp.int32))
    return padded[:, :D].reshape(B, H, D)
